# baseline (speedup 1.0000x reference)
import jax
import jax.numpy as jnp
from jax.experimental import pallas as pl
from jax.experimental.pallas import tpu as pltpu

B = 8
N = 8192
F = 512
L = 500
D = 128
NUM_CLASSES = 2
BUFFER = 1024

T = 8192
NT = N // T
STEPS = B * NT
RB = BUFFER // STEPS
NV = RB // 8


def _fused_kernel(x_ref, W1_ref, b1_ref, Wa1_ref, ba1_ref, Wa2t_ref,
                  Wc_ref, bc_ref, *refs):
    reh_refs = refs[:NV]
    logits_ref, newreh_ref, acc_ref, m_ref, s_ref, bf_ref = refs[NV:]
    b = pl.program_id(0)
    n = pl.program_id(1)
    t = b * NT + n

    @pl.when(n == 0)
    def _init():
        m_ref[0, 0] = -1e30
        s_ref[0, 0] = 0.0
        acc_ref[...] = jnp.zeros_like(acc_ref)

    x_t = x_ref[0]
    H = jnp.maximum(jnp.dot(x_t.astype(jnp.bfloat16), W1_ref[...],
                            preferred_element_type=jnp.float32)
                    + b1_ref[0], 0.0)
    Hb = H.astype(jnp.bfloat16)
    a = jnp.maximum(jnp.dot(Hb, Wa1_ref[...],
                            preferred_element_type=jnp.float32)
                    + ba1_ref[0], 0.0)
    sc = jax.lax.dot_general(Wa2t_ref[...], a, (((1,), (1,)), ((), ())),
                             preferred_element_type=jnp.float32)

    m_old = m_ref[0, 0]
    m_new = jnp.maximum(m_old, jnp.max(sc))
    corr = jnp.exp(m_old - m_new)
    p = jnp.exp(sc - m_new)
    m_ref[0, 0] = m_new
    s_ref[0, 0] = s_ref[0, 0] * corr + jnp.sum(p)
    acc_ref[...] = acc_ref[...] * corr + jnp.dot(
        p, H, preferred_element_type=jnp.float32)

    @pl.when(n == NT - 1)
    def _finish_bag():
        bf = acc_ref[...] / s_ref[0, 0]
        bf_ref[pl.ds(b, 1), :] = bf
        logits_ref[0] = jnp.dot(bf, Wc_ref[...],
                                preferred_element_type=jnp.float32) + bc_ref[...]

    @pl.when(t < STEPS - 1)
    def _copy_reh():
        newreh_ref[0:8, :] = reh_refs[0][...]

    @pl.when(t == STEPS - 1)
    def _write_bagfeats():
        newreh_ref[0:8, :] = bf_ref[...]

    for j in range(1, NV):
        newreh_ref[8 * j:8 * (j + 1), :] = reh_refs[j][...]


def kernel(x, W1, b1, Wa1, ba1, Wa2, ba2, Wc, bc, rehearsal):
    reh_flat = rehearsal.reshape(BUFFER, L)

    grid = (B, NT)
    in_specs = [
        pl.BlockSpec((1, T, F), lambda b, n: (b, n, 0)),
        pl.BlockSpec((F, L), lambda b, n: (0, 0)),
        pl.BlockSpec((1, L), lambda b, n: (0, 0)),
        pl.BlockSpec((L, D), lambda b, n: (0, 0)),
        pl.BlockSpec((1, D), lambda b, n: (0, 0)),
        pl.BlockSpec((1, D), lambda b, n: (0, 0)),
        pl.BlockSpec((L, NUM_CLASSES), lambda b, n: (0, 0)),
        pl.BlockSpec((1, NUM_CLASSES), lambda b, n: (0, 0)),
    ] + [
        pl.BlockSpec(
            (8, L),
            (lambda j: (lambda b, n: (jnp.maximum(
                NV * ((b * NT + n + 1) % STEPS) - 1 + j, 0), 0)))(j))
        for j in range(NV)
    ]
    out_specs = [
        pl.BlockSpec((1, 1, NUM_CLASSES), lambda b, n: (b, 0, 0)),
        pl.BlockSpec((RB, L), lambda b, n: ((b * NT + n + 1) % STEPS, 0)),
    ]
    out_shapes = [
        jax.ShapeDtypeStruct((B, 1, NUM_CLASSES), jnp.float32),
        jax.ShapeDtypeStruct((BUFFER, L), jnp.float32),
    ]
    scratch_shapes = [
        pltpu.VMEM((1, L), jnp.float32),
        pltpu.SMEM((1, 1), jnp.float32),
        pltpu.SMEM((1, 1), jnp.float32),
        pltpu.VMEM((B, L), jnp.float32),
    ]

    logits, newreh = pl.pallas_call(
        _fused_kernel,
        grid=grid,
        in_specs=in_specs,
        out_specs=out_specs,
        out_shape=out_shapes,
        scratch_shapes=scratch_shapes,
        compiler_params=pltpu.CompilerParams(
            dimension_semantics=("arbitrary", "arbitrary"),
        ),
    )(x, W1.astype(jnp.bfloat16), b1.reshape(1, L),
      Wa1.astype(jnp.bfloat16), ba1.reshape(1, D), Wa2.reshape(1, D),
      Wc, bc.reshape(1, NUM_CLASSES), *([reh_flat] * NV))

    return (logits.reshape(B, NUM_CLASSES),
            newreh.reshape(NUM_CLASSES, BUFFER // NUM_CLASSES, L))


# speedup vs baseline: 1.0327x; 1.0327x over previous
import jax
import jax.numpy as jnp
from jax.experimental import pallas as pl
from jax.experimental.pallas import tpu as pltpu

B = 8
N = 8192
F = 512
L = 500
D = 128
NUM_CLASSES = 2
BUFFER = 1024

T = 8192
NT = N // T
STEPS = B * NT
RB = BUFFER // STEPS
NV = RB // 8


def _fused_kernel(x_ref, W1_ref, b1_ref, Wa1_ref, ba1_ref, Wa2t_ref,
                  Wc_ref, bc_ref, *refs):
    reh_refs = refs[:NV]
    logits_ref, newreh_ref, acc_ref, m_ref, s_ref, bf_ref = refs[NV:]
    b = pl.program_id(0)
    n = pl.program_id(1)
    t = b * NT + n

    @pl.when(n == 0)
    def _init():
        m_ref[0, 0] = -1e30
        s_ref[0, 0] = 0.0
        acc_ref[...] = jnp.zeros_like(acc_ref)

    TH = T // 2
    Hs, scs = [], []
    for h in range(2):
        x_h = x_ref[0, pl.ds(h * TH, TH), :]
        H = jnp.maximum(jnp.dot(x_h.astype(jnp.bfloat16), W1_ref[...],
                                preferred_element_type=jnp.float32)
                        + b1_ref[0], 0.0)
        a = jnp.maximum(jnp.dot(H.astype(jnp.bfloat16), Wa1_ref[...],
                                preferred_element_type=jnp.float32)
                        + ba1_ref[0], 0.0)
        Hs.append(H)
        scs.append(jax.lax.dot_general(
            Wa2t_ref[...], a, (((1,), (1,)), ((), ())),
            preferred_element_type=jnp.float32))

    sc = jnp.concatenate(scs, axis=1)
    m_old = m_ref[0, 0]
    m_new = jnp.maximum(m_old, jnp.max(sc))
    corr = jnp.exp(m_old - m_new)
    p = jnp.exp(sc - m_new)
    m_ref[0, 0] = m_new
    s_ref[0, 0] = s_ref[0, 0] * corr + jnp.sum(p)
    acc_ref[...] = (acc_ref[...] * corr
                    + jnp.dot(p[:, :TH], Hs[0],
                              preferred_element_type=jnp.float32)
                    + jnp.dot(p[:, TH:], Hs[1],
                              preferred_element_type=jnp.float32))

    @pl.when(n == NT - 1)
    def _finish_bag():
        bf = acc_ref[...] / s_ref[0, 0]
        bf_ref[pl.ds(b, 1), :] = bf
        logits_ref[0] = jnp.dot(bf, Wc_ref[...],
                                preferred_element_type=jnp.float32) + bc_ref[...]

    @pl.when(t < STEPS - 1)
    def _copy_reh():
        newreh_ref[0:8, :] = reh_refs[0][...]

    @pl.when(t == STEPS - 1)
    def _write_bagfeats():
        newreh_ref[0:8, :] = bf_ref[...]

    for j in range(1, NV):
        newreh_ref[8 * j:8 * (j + 1), :] = reh_refs[j][...]


def kernel(x, W1, b1, Wa1, ba1, Wa2, ba2, Wc, bc, rehearsal):
    reh_flat = rehearsal.reshape(BUFFER, L)

    grid = (B, NT)
    in_specs = [
        pl.BlockSpec((1, T, F), lambda b, n: (b, n, 0)),
        pl.BlockSpec((F, L), lambda b, n: (0, 0)),
        pl.BlockSpec((1, L), lambda b, n: (0, 0)),
        pl.BlockSpec((L, D), lambda b, n: (0, 0)),
        pl.BlockSpec((1, D), lambda b, n: (0, 0)),
        pl.BlockSpec((1, D), lambda b, n: (0, 0)),
        pl.BlockSpec((L, NUM_CLASSES), lambda b, n: (0, 0)),
        pl.BlockSpec((1, NUM_CLASSES), lambda b, n: (0, 0)),
    ] + [
        pl.BlockSpec(
            (8, L),
            (lambda j: (lambda b, n: (jnp.maximum(
                NV * ((b * NT + n + 1) % STEPS) - 1 + j, 0), 0)))(j))
        for j in range(NV)
    ]
    out_specs = [
        pl.BlockSpec((1, 1, NUM_CLASSES), lambda b, n: (b, 0, 0)),
        pl.BlockSpec((RB, L), lambda b, n: ((b * NT + n + 1) % STEPS, 0)),
    ]
    out_shapes = [
        jax.ShapeDtypeStruct((B, 1, NUM_CLASSES), jnp.float32),
        jax.ShapeDtypeStruct((BUFFER, L), jnp.float32),
    ]
    scratch_shapes = [
        pltpu.VMEM((1, L), jnp.float32),
        pltpu.SMEM((1, 1), jnp.float32),
        pltpu.SMEM((1, 1), jnp.float32),
        pltpu.VMEM((B, L), jnp.float32),
    ]

    logits, newreh = pl.pallas_call(
        _fused_kernel,
        grid=grid,
        in_specs=in_specs,
        out_specs=out_specs,
        out_shape=out_shapes,
        scratch_shapes=scratch_shapes,
        compiler_params=pltpu.CompilerParams(
            dimension_semantics=("arbitrary", "arbitrary"),
        ),
    )(x, W1.astype(jnp.bfloat16), b1.reshape(1, L),
      Wa1.astype(jnp.bfloat16), ba1.reshape(1, D), Wa2.reshape(1, D),
      Wc, bc.reshape(1, NUM_CLASSES), *([reh_flat] * NV))

    return (logits.reshape(B, NUM_CLASSES),
            newreh.reshape(NUM_CLASSES, BUFFER // NUM_CLASSES, L))
